# baseline (speedup 1.0000x reference)
import functools

import jax
import jax.numpy as jnp
from jax import lax
from jax.experimental import pallas as pl
from jax.experimental.pallas import tpu as pltpu
from jax.experimental.pallas import tpu_sc as plsc

_NC = 2
_NS = 16
_NW = _NC * _NS

_CHUNK = 128
_NBUF = 4
_NQ = 2


@functools.partial(jax.jit, static_argnames=("b_total", "d"))
def _sc_gather(idx_flat, table, *, b_total, d):
    rows_per_w = b_total // _NW
    n_chunks = rows_per_w // _CHUNK
    n_outer = n_chunks // _NBUF

    mesh = plsc.VectorSubcoreMesh(
        core_axis_name="c", subcore_axis_name="s",
        num_cores=_NC, num_subcores=_NS,
    )

    @functools.partial(
        pl.kernel,
        out_type=jax.ShapeDtypeStruct((b_total, d), jnp.float32),
        mesh=mesh,
        scratch_types=[
            pltpu.VMEM((rows_per_w,), jnp.int32),
            [pltpu.VMEM((_CHUNK, d), jnp.float32)] * _NBUF,
            pltpu.VMEM_SHARED((_NS, _NQ, _CHUNK, d), jnp.float32),
            [pltpu.SemaphoreType.DMA] * _NBUF,
            [pltpu.SemaphoreType.DMA] * _NBUF,
            [pltpu.SemaphoreType.DMA] * _NQ,
        ],
    )
    def k(idx_hbm, table_hbm, out_hbm, idx_v, rows, spm, gsems, csems, dsems):
        sid = lax.axis_index("s")
        wid = sid * _NC + lax.axis_index("c")
        row_base = wid * rows_per_w

        pltpu.sync_copy(idx_hbm.at[pl.ds(row_base, rows_per_w)], idx_v)

        def fire_gather(g, b):
            pltpu.async_copy(
                table_hbm.at[idx_v.at[pl.ds(g * _CHUNK, _CHUNK)]],
                rows[b], gsems[b])

        def drain_gather(b):
            pltpu.make_async_copy(
                out_hbm.at[pl.ds(0, _CHUNK)], rows[b], gsems[b]).wait()

        def fire_copy(b, q):
            pltpu.async_copy(rows[b], spm.at[sid, q], csems[b])

        def drain_copy(b, q):
            pltpu.make_async_copy(rows[b], spm.at[sid, q], csems[b]).wait()

        def fire_dma(g, q):
            pltpu.async_copy(
                spm.at[sid, q],
                out_hbm.at[pl.ds(row_base + g * _CHUNK, _CHUNK)], dsems[q])

        def drain_dma(q):
            pltpu.make_async_copy(
                spm.at[sid, q], out_hbm.at[pl.ds(0, _CHUNK)], dsems[q]).wait()

        def step(p, b, carry):
            g = p * _NBUF + b
            b2 = (b - 2) % _NBUF
            b3 = (b - 3) % _NBUF
            q2 = b % _NQ

            fire_gather(g, b)

            @pl.when(g >= 2)
            def _():
                @pl.when(g >= 4)
                def _():
                    drain_dma(q2)
                drain_gather(b2)
                fire_copy(b2, q2)

            @pl.when(g >= 3)
            def _():
                drain_copy(b3, (b - 3) % _NQ)
                fire_dma(g - 3, (b - 3) % _NQ)

            return carry

        def outer(p, carry):
            for b in range(_NBUF):
                carry = step(p, b, carry)
            return carry

        lax.fori_loop(0, n_outer, outer, 0)

        n = n_chunks
        for gv in (n, n + 1):
            drain_dma(gv % _NQ)
            drain_gather((gv - 2) % _NBUF)
            fire_copy((gv - 2) % _NBUF, gv % _NQ)
            drain_copy((gv - 3) % _NBUF, (gv - 3) % _NQ)
            fire_dma(gv - 3, (gv - 3) % _NQ)
        drain_copy((n - 1) % _NBUF, (n - 1) % _NQ)
        fire_dma(n - 1, (n - 1) % _NQ)
        drain_dma((n - 2) % _NQ)
        drain_dma((n - 1) % _NQ)

    return k(idx_flat, table)


def kernel(x, table):
    b, s = x.shape
    v, d = table.shape
    out = _sc_gather(x.reshape(b * s), table, b_total=b * s, d=d)
    return out.reshape(b, s, d)


# speedup vs baseline: 1.0072x; 1.0072x over previous
import functools

import jax
import jax.numpy as jnp
from jax import lax
from jax.experimental import pallas as pl
from jax.experimental.pallas import tpu as pltpu
from jax.experimental.pallas import tpu_sc as plsc

_NC = 2
_NS = 16
_NW = _NC * _NS

_CHUNK = 128
_NBUF = 4
_NQ = 2


@functools.partial(jax.jit, static_argnames=("b_total", "d"))
def _sc_gather(idx_flat, table, *, b_total, d):
    rows_per_w = b_total // _NW
    n_chunks = rows_per_w // _CHUNK
    n_outer = n_chunks // _NBUF

    mesh = plsc.VectorSubcoreMesh(
        core_axis_name="c", subcore_axis_name="s",
        num_cores=_NC, num_subcores=_NS,
    )

    @functools.partial(
        pl.kernel,
        out_type=jax.ShapeDtypeStruct((b_total, d), jnp.float32),
        mesh=mesh,
        scratch_types=[
            pltpu.VMEM((rows_per_w,), jnp.int32),
            [pltpu.VMEM((_CHUNK, d), jnp.float32)] * _NBUF,
            pltpu.VMEM_SHARED((_NS, _NQ, _CHUNK, d), jnp.float32),
            [pltpu.SemaphoreType.DMA] * _NBUF,
            [pltpu.SemaphoreType.DMA] * _NBUF,
            [pltpu.SemaphoreType.DMA] * _NQ,
        ],
    )
    def k(idx_hbm, table_hbm, out_hbm, idx_v, rows, spm, gsems, csems, dsems):
        sid = lax.axis_index("s")
        wid = sid * _NC + lax.axis_index("c")
        row_base = wid * rows_per_w

        pltpu.sync_copy(idx_hbm.at[pl.ds(row_base, rows_per_w)], idx_v)

        def fire_gather(g, b):
            pltpu.async_copy(
                table_hbm.at[idx_v.at[pl.ds(g * _CHUNK, _CHUNK)]],
                rows[b], gsems[b])

        def drain_gather(b):
            pltpu.make_async_copy(
                out_hbm.at[pl.ds(0, _CHUNK)], rows[b], gsems[b]).wait()

        def fire_copy(b, q):
            pltpu.async_copy(rows[b], spm.at[sid, q], csems[b])

        def drain_copy(b, q):
            pltpu.make_async_copy(rows[b], spm.at[sid, q], csems[b]).wait()

        def fire_dma(g, q):
            pltpu.async_copy(
                spm.at[sid, q],
                out_hbm.at[pl.ds(row_base + g * _CHUNK, _CHUNK)], dsems[q])

        def drain_dma(q):
            pltpu.make_async_copy(
                spm.at[sid, q], out_hbm.at[pl.ds(0, _CHUNK)], dsems[q]).wait()

        def step(p, b, carry):
            g = p * _NBUF + b
            b3 = (b - 3) % _NBUF
            q3 = (b - 3) % _NQ

            @pl.when(g >= 4)
            def _():
                drain_copy(b, b % _NQ)
                fire_dma(g - 4, b % _NQ)

            fire_gather(g, b)

            @pl.when(g >= 3)
            def _():
                @pl.when(g >= 5)
                def _():
                    drain_dma(q3)
                drain_gather(b3)
                fire_copy(b3, q3)

            return carry

        def outer(p, carry):
            for b in range(_NBUF):
                carry = step(p, b, carry)
            return carry

        lax.fori_loop(0, n_outer, outer, 0)

        n = n_chunks
        for gv in (n, n + 1, n + 2):
            drain_copy(gv % _NBUF, gv % _NQ)
            fire_dma(gv - 4, gv % _NQ)
            drain_dma((gv - 3) % _NQ)
            drain_gather((gv - 3) % _NBUF)
            fire_copy((gv - 3) % _NBUF, (gv - 3) % _NQ)
        drain_copy((n - 1) % _NBUF, (n - 1) % _NQ)
        fire_dma(n - 1, (n - 1) % _NQ)
        drain_dma((n - 2) % _NQ)
        drain_dma((n - 1) % _NQ)

    return k(idx_flat, table)


def kernel(x, table):
    b, s = x.shape
    v, d = table.shape
    out = _sc_gather(x.reshape(b * s), table, b_total=b * s, d=d)
    return out.reshape(b, s, d)
